# baseline (speedup 1.0000x reference)
import functools

import jax
import jax.numpy as jnp
from jax import lax
from jax.experimental import pallas as pl
from jax.experimental.pallas import tpu as pltpu
from jax.experimental.pallas import tpu_sc as plsc

N = 10000
E = 320000
H = 128
NC = 2
NS = 16
NW = NC * NS
KB = 80
BPW = 125
EP = NW * BPW * KB
ROWS_PER_TILE = 632
NP = NS * ROWS_PER_TILE

_mesh = plsc.VectorSubcoreMesh(core_axis_name="c", subcore_axis_name="s")


def _worker_id():
    return lax.axis_index("s") * NC + lax.axis_index("c")


@functools.partial(
    pl.kernel,
    out_type=jax.ShapeDtypeStruct((NW, NP), jnp.float32),
    mesh=_mesh,
    scratch_types=[
        pltpu.VMEM((BPW, KB), jnp.int32),
        pltpu.VMEM((NP,), jnp.float32),
    ],
    compiler_params=pltpu.CompilerParams(needs_layout_passes=False),
)
def _sc_deg(dst_hbm, out_hbm, idx_v, deg_v):
    wid = _worker_id()
    zero16 = jnp.zeros((16,), jnp.float32)
    ones16 = jnp.ones((16,), jnp.float32)

    def zero_body(i, _):
        deg_v[pl.ds(i * 16, 16)] = zero16
        return _

    lax.fori_loop(0, NP // 16, zero_body, None)
    pltpu.sync_copy(dst_hbm.at[wid], idx_v)

    def batch_body(t, _):
        for j in range(KB // 16):
            iv = idx_v[t, pl.ds(j * 16, 16)]
            plsc.addupdate_scatter(deg_v, [iv], ones16)
        return _

    lax.fori_loop(0, BPW, batch_body, None)
    pltpu.sync_copy(deg_v, out_hbm.at[wid])


HW = H // 2


@functools.partial(
    pl.kernel,
    out_type=jax.ShapeDtypeStruct((NC, NP, HW), jnp.float32),
    mesh=_mesh,
    scratch_types=[
        pltpu.VMEM((BPW, KB), jnp.int32),
        pltpu.VMEM((BPW, KB), jnp.int32),
        pltpu.VMEM((2, KB, HW), jnp.float32),
        pltpu.VMEM((ROWS_PER_TILE, HW), jnp.float32),
        pltpu.VMEM_SHARED((NP, HW), jnp.float32),
        pltpu.SemaphoreType.DMA,
        pltpu.SemaphoreType.DMA,
    ],
    compiler_params=pltpu.CompilerParams(use_tc_tiling_on_sc=False),
)
def _sc_seg(src_hbm, dst_hbm, u_hbm, out_hbm, si_v, di_v, rows_v, buf_v,
            acc_sh, sem_g, sem_s):
    c = lax.axis_index("c")
    s = lax.axis_index("s")
    wid = _worker_id()
    zero16 = jnp.zeros((16,), jnp.float32)

    def zero_body(i, _):
        for j in range(HW // 16):
            buf_v[i, pl.ds(j * 16, 16)] = zero16
        return _

    lax.fori_loop(0, ROWS_PER_TILE, zero_body, None)
    pltpu.sync_copy(buf_v, acc_sh.at[pl.ds(s * ROWS_PER_TILE, ROWS_PER_TILE)])
    plsc.subcore_barrier()

    pltpu.sync_copy(src_hbm.at[wid], si_v)
    pltpu.sync_copy(dst_hbm.at[wid], di_v)
    pltpu.async_copy(u_hbm.at[si_v.at[0]], rows_v.at[0], sem_g)

    def batch_body(t, _):
        cur = lax.rem(t, 2)
        nxt = lax.rem(t + 1, 2)
        pltpu.make_async_copy(u_hbm.at[si_v.at[0]], rows_v.at[cur],
                              sem_g).wait()

        @pl.when(t + 1 < BPW)
        def _issue():
            @pl.when(t >= 1)
            def _drain():
                pltpu.make_async_copy(rows_v.at[0], acc_sh.at[di_v.at[0]],
                                      sem_s).wait()

            pltpu.async_copy(u_hbm.at[si_v.at[t + 1]], rows_v.at[nxt], sem_g)

        pltpu.async_copy(rows_v.at[cur], acc_sh.at[di_v.at[t]], sem_s,
                         add=True)
        return _

    lax.fori_loop(0, BPW, batch_body, None)
    pltpu.make_async_copy(rows_v.at[0], acc_sh.at[di_v.at[0]], sem_s).wait()
    pltpu.make_async_copy(rows_v.at[0], acc_sh.at[di_v.at[0]], sem_s).wait()
    plsc.subcore_barrier()
    pltpu.sync_copy(acc_sh.at[pl.ds(s * ROWS_PER_TILE, ROWS_PER_TILE)], buf_v)
    pltpu.sync_copy(buf_v,
                    out_hbm.at[c, pl.ds(s * ROWS_PER_TILE, ROWS_PER_TILE)])


def _seg_sum(src, dst, u):
    lo = _sc_seg(src, dst, u[:, :HW])
    hi = _sc_seg(src, dst, u[:, HW:])
    return lo, hi


def _make_sc_gather(bpw, ep):
    @functools.partial(
        pl.kernel,
        out_type=(
            jax.ShapeDtypeStruct((ep, H), jnp.float32),
            jax.ShapeDtypeStruct((ep, H), jnp.float32),
        ),
        mesh=_mesh,
        scratch_types=[
            pltpu.VMEM((bpw, KB), jnp.int32),
            pltpu.VMEM((bpw, KB), jnp.int32),
            pltpu.VMEM((2, KB, H), jnp.float32),
            pltpu.VMEM((2, KB, H), jnp.float32),
            pltpu.SemaphoreType.DMA,
            pltpu.SemaphoreType.DMA,
            pltpu.SemaphoreType.DMA,
        ],
    )
    def _sc_gather(src_hbm, dst_hbm, h_hbm, hs_hbm, hd_hbm, si_v, di_v,
                   ra_v, rb_v, sem_a, sem_b, sem_w):
        wid = _worker_id()

        pltpu.sync_copy(src_hbm.at[wid], si_v)
        pltpu.sync_copy(dst_hbm.at[wid], di_v)
        pltpu.async_copy(h_hbm.at[si_v.at[0]], ra_v.at[0], sem_a)
        pltpu.async_copy(h_hbm.at[di_v.at[0]], rb_v.at[0], sem_b)

        def batch_body(t, _):
            cur = lax.rem(t, 2)
            nxt = lax.rem(t + 1, 2)
            off = pl.multiple_of((wid * bpw + t) * KB, KB)
            pltpu.make_async_copy(h_hbm.at[si_v.at[0]], ra_v.at[cur],
                                  sem_a).wait()
            pltpu.make_async_copy(h_hbm.at[di_v.at[0]], rb_v.at[cur],
                                  sem_b).wait()

            @pl.when(t + 1 < bpw)
            def _issue():
                @pl.when(t >= 1)
                def _drain():
                    pltpu.make_async_copy(ra_v.at[0],
                                          hs_hbm.at[pl.ds(0, KB)],
                                          sem_w).wait()
                    pltpu.make_async_copy(rb_v.at[0],
                                          hd_hbm.at[pl.ds(0, KB)],
                                          sem_w).wait()

                pltpu.async_copy(h_hbm.at[si_v.at[t + 1]], ra_v.at[nxt],
                                 sem_a)
                pltpu.async_copy(h_hbm.at[di_v.at[t + 1]], rb_v.at[nxt],
                                 sem_b)

            pltpu.async_copy(ra_v.at[cur], hs_hbm.at[pl.ds(off, KB)], sem_w)
            pltpu.async_copy(rb_v.at[cur], hd_hbm.at[pl.ds(off, KB)], sem_w)
            return _

        lax.fori_loop(0, bpw, batch_body, None)
        for _ in range(2):
            pltpu.make_async_copy(ra_v.at[0], hs_hbm.at[pl.ds(0, KB)],
                                  sem_w).wait()
            pltpu.make_async_copy(rb_v.at[0], hd_hbm.at[pl.ds(0, KB)],
                                  sem_w).wait()

    return _sc_gather


BPW_A = 64
BPW_B = BPW - BPW_A
E_A = NW * BPW_A * KB
E_B = E - E_A
_sc_gather_a = _make_sc_gather(BPW_A, E_A)
_sc_gather_b = _make_sc_gather(BPW_B, E_B)


def _tc_mm_body(degp_ref, x_ref, w_ref, u_ref, dis_ref):
    deg = jnp.sum(degp_ref[...], axis=0)[:N] + 1.0
    dis = (1.0 / jnp.sqrt(deg))[:, None]
    dis_ref[...] = dis
    h = jnp.dot(x_ref[...], w_ref[...], preferred_element_type=jnp.float32)
    u_ref[...] = h * dis


def _tc_mm(degp, x, w):
    return pl.pallas_call(
        _tc_mm_body,
        out_shape=(jax.ShapeDtypeStruct((N, H), jnp.float32),
                   jax.ShapeDtypeStruct((N, 1), jnp.float32)),
    )(degp, x, w)


def _bn_relu(slo_ref, shi_ref, u_ref, dis_ref, g_ref, be_ref):
    Slo = slo_ref[...]
    Shi = shi_ref[...]
    S = jnp.concatenate([Slo[0, :N] + Slo[1, :N], Shi[0, :N] + Shi[1, :N]],
                        axis=-1)
    a = (S + u_ref[...]) * dis_ref[...]
    m = jnp.mean(a, axis=0, keepdims=True)
    v = jnp.mean((a - m) ** 2, axis=0, keepdims=True)
    hn = (a - m) * jax.lax.rsqrt(v + 1e-5) * g_ref[...] + be_ref[...]
    return jnp.maximum(hn, 0.0)


def _tc_bnmm_body(slo_ref, shi_ref, u_ref, dis_ref, g_ref, be_ref, w_ref,
                  o_ref):
    hh = _bn_relu(slo_ref, shi_ref, u_ref, dis_ref, g_ref, be_ref)
    o_ref[...] = jnp.dot(hh, w_ref[...],
                         preferred_element_type=jnp.float32) * dis_ref[...]


def _tc_bnmm(Slo, Shi, u, dis, g, be, w):
    return pl.pallas_call(
        _tc_bnmm_body,
        out_shape=jax.ShapeDtypeStruct((N, H), jnp.float32),
    )(Slo, Shi, u, dis, g, be, w)


def _tc_bn_body(slo_ref, shi_ref, u_ref, dis_ref, g_ref, be_ref, o_ref):
    o_ref[...] = _bn_relu(slo_ref, shi_ref, u_ref, dis_ref, g_ref, be_ref)


def _tc_bn(Slo, Shi, u, dis, g, be):
    return pl.pallas_call(
        _tc_bn_body,
        out_shape=jax.ShapeDtypeStruct((N, H), jnp.float32),
    )(Slo, Shi, u, dis, g, be)


def _bdot(a, b):
    return jnp.dot(a.astype(jnp.bfloat16), b.astype(jnp.bfloat16),
                   preferred_element_type=jnp.float32)


def _tc_mlp_body(hs_ref, hd_ref, ea_ref, wa_ref, wb_ref, we_ref, bc1_ref,
                 w2_ref, bc2_ref, w3_ref, bc3_ref, w4_ref, bc4_ref, o_ref):
    z = (_bdot(hs_ref[...], wa_ref[...])
         + _bdot(hd_ref[...], wb_ref[...])
         + jnp.dot(ea_ref[...], we_ref[...], preferred_element_type=jnp.float32)
         + bc1_ref[...])
    z = jnp.maximum(z, 0.0)
    z = jnp.maximum(_bdot(z, w2_ref[...]) + bc2_ref[...], 0.0)
    z = jnp.maximum(_bdot(z, w3_ref[...]) + bc3_ref[...], 0.0)
    o_ref[...] = jnp.dot(z, w4_ref[...],
                         preferred_element_type=jnp.float32) + bc4_ref[...]


def _tc_mlp(hs, hd, ea, wa, wb, we, bc1, w2, bc2, w3, bc3, w4, bc4, e, eb):
    grid = (e // eb,)
    row_spec = lambda w: pl.BlockSpec((eb, w), lambda i: (i, 0))

    def wspec(a):
        return pl.BlockSpec(a.shape, lambda i: tuple(0 for _ in a.shape))

    return pl.pallas_call(
        _tc_mlp_body,
        grid=grid,
        in_specs=[
            row_spec(H), row_spec(H), row_spec(16),
            wspec(wa), wspec(wb), wspec(we), wspec(bc1),
            wspec(w2), wspec(bc2), wspec(w3), wspec(bc3),
            wspec(w4), wspec(bc4),
        ],
        out_specs=pl.BlockSpec((eb, 2), lambda i: (i, 0)),
        out_shape=jax.ShapeDtypeStruct((e, 2), jnp.float32),
    )(hs, hd, ea, wa, wb, we, bc1, w2, bc2, w3, bc3, w4, bc4)


def kernel(x, edge_index, edge_attr, W1, b1, g1, be1, W2, b2, g2, be2,
           W3, b3, g3, be3, Wc1, bc1, Wc2, bc2, Wc3, bc3, Wc4, bc4):
    src = edge_index[0].reshape(NW, BPW, KB)
    dst = edge_index[1].reshape(NW, BPW, KB)

    degp = _sc_deg(dst)

    u1, dis = _tc_mm(degp, x, W1)
    S1lo, S1hi = _seg_sum(src, dst, u1)
    u2 = _tc_bnmm(S1lo, S1hi, u1, dis, g1, be1, W2)
    S2lo, S2hi = _seg_sum(src, dst, u2)
    u3 = _tc_bnmm(S2lo, S2hi, u2, dis, g2, be2, W3)
    S3lo, S3hi = _seg_sum(src, dst, u3)
    h3 = _tc_bn(S3lo, S3hi, u3, dis, g3, be3)

    src_a = edge_index[0][:E_A].reshape(NW, BPW_A, KB)
    dst_a = edge_index[1][:E_A].reshape(NW, BPW_A, KB)
    src_b = edge_index[0][E_A:].reshape(NW, BPW_B, KB)
    dst_b = edge_index[1][E_A:].reshape(NW, BPW_B, KB)

    Wc1a = Wc1[:H]
    Wc1b = Wc1[H:2 * H]
    Wc1e = Wc1[2 * H:]

    hs_a, hd_a = _sc_gather_a(src_a, dst_a, h3)
    hs_b, hd_b = _sc_gather_b(src_b, dst_b, h3)
    out_a = _tc_mlp(hs_a, hd_a, edge_attr[:E_A], Wc1a, Wc1b, Wc1e, bc1,
                    Wc2, bc2, Wc3, bc3, Wc4, bc4, E_A, 8192)
    out_b = _tc_mlp(hs_b, hd_b, edge_attr[E_A:], Wc1a, Wc1b, Wc1e, bc1,
                    Wc2, bc2, Wc3, bc3, Wc4, bc4, E_B, 7808)
    return jnp.concatenate([out_a, out_b], axis=0)


# speedup vs baseline: 1.0111x; 1.0111x over previous
import functools

import jax
import jax.numpy as jnp
from jax import lax
from jax.experimental import pallas as pl
from jax.experimental.pallas import tpu as pltpu
from jax.experimental.pallas import tpu_sc as plsc

N = 10000
E = 320000
H = 128
NC = 2
NS = 16
NW = NC * NS
KB = 80
BPW = 125
EP = NW * BPW * KB
ROWS_PER_TILE = 632
NP = NS * ROWS_PER_TILE

_mesh = plsc.VectorSubcoreMesh(core_axis_name="c", subcore_axis_name="s")


def _worker_id():
    return lax.axis_index("s") * NC + lax.axis_index("c")


@functools.partial(
    pl.kernel,
    out_type=jax.ShapeDtypeStruct((NW, NP), jnp.float32),
    mesh=_mesh,
    scratch_types=[
        pltpu.VMEM((BPW, KB), jnp.int32),
        pltpu.VMEM((NP,), jnp.float32),
    ],
    compiler_params=pltpu.CompilerParams(needs_layout_passes=False),
)
def _sc_deg(dst_hbm, out_hbm, idx_v, deg_v):
    wid = _worker_id()
    zero16 = jnp.zeros((16,), jnp.float32)
    ones16 = jnp.ones((16,), jnp.float32)

    def zero_body(i, _):
        deg_v[pl.ds(i * 16, 16)] = zero16
        return _

    lax.fori_loop(0, NP // 16, zero_body, None)
    pltpu.sync_copy(dst_hbm.at[wid], idx_v)

    def batch_body(t, _):
        for j in range(KB // 16):
            iv = idx_v[t, pl.ds(j * 16, 16)]
            plsc.addupdate_scatter(deg_v, [iv], ones16)
        return _

    lax.fori_loop(0, BPW, batch_body, None)
    pltpu.sync_copy(deg_v, out_hbm.at[wid])


HW = H // 2


@functools.partial(
    pl.kernel,
    out_type=jax.ShapeDtypeStruct((NC, NP, HW), jnp.float32),
    mesh=_mesh,
    scratch_types=[
        pltpu.VMEM((BPW, KB), jnp.int32),
        pltpu.VMEM((BPW, KB), jnp.int32),
        pltpu.VMEM((2, KB, HW), jnp.float32),
        pltpu.VMEM((ROWS_PER_TILE, HW), jnp.float32),
        pltpu.VMEM_SHARED((NP, HW), jnp.float32),
        pltpu.SemaphoreType.DMA,
        pltpu.SemaphoreType.DMA,
    ],
    compiler_params=pltpu.CompilerParams(use_tc_tiling_on_sc=False),
)
def _sc_seg(src_hbm, dst_hbm, u_hbm, out_hbm, si_v, di_v, rows_v, buf_v,
            acc_sh, sem_g, sem_s):
    c = lax.axis_index("c")
    s = lax.axis_index("s")
    wid = _worker_id()
    zero16 = jnp.zeros((16,), jnp.float32)

    def zero_body(i, _):
        for j in range(HW // 16):
            buf_v[i, pl.ds(j * 16, 16)] = zero16
        return _

    lax.fori_loop(0, ROWS_PER_TILE, zero_body, None)
    pltpu.sync_copy(buf_v, acc_sh.at[pl.ds(s * ROWS_PER_TILE, ROWS_PER_TILE)])
    plsc.subcore_barrier()

    pltpu.sync_copy(src_hbm.at[wid], si_v)
    pltpu.sync_copy(dst_hbm.at[wid], di_v)
    pltpu.async_copy(u_hbm.at[si_v.at[0]], rows_v.at[0], sem_g)

    def batch_body(t, _):
        cur = lax.rem(t, 2)
        nxt = lax.rem(t + 1, 2)
        pltpu.make_async_copy(u_hbm.at[si_v.at[0]], rows_v.at[cur],
                              sem_g).wait()

        @pl.when(t + 1 < BPW)
        def _issue():
            @pl.when(t >= 1)
            def _drain():
                pltpu.make_async_copy(rows_v.at[0], acc_sh.at[di_v.at[0]],
                                      sem_s).wait()

            pltpu.async_copy(u_hbm.at[si_v.at[t + 1]], rows_v.at[nxt], sem_g)

        pltpu.async_copy(rows_v.at[cur], acc_sh.at[di_v.at[t]], sem_s,
                         add=True)
        return _

    lax.fori_loop(0, BPW, batch_body, None)
    pltpu.make_async_copy(rows_v.at[0], acc_sh.at[di_v.at[0]], sem_s).wait()
    pltpu.make_async_copy(rows_v.at[0], acc_sh.at[di_v.at[0]], sem_s).wait()
    plsc.subcore_barrier()
    pltpu.sync_copy(acc_sh.at[pl.ds(s * ROWS_PER_TILE, ROWS_PER_TILE)], buf_v)
    pltpu.sync_copy(buf_v,
                    out_hbm.at[c, pl.ds(s * ROWS_PER_TILE, ROWS_PER_TILE)])


def _seg_sum(src, dst, u):
    lo = _sc_seg(src, dst, u[:, :HW])
    hi = _sc_seg(src, dst, u[:, HW:])
    return lo, hi


def _make_sc_gather(bpw, ep):
    @functools.partial(
        pl.kernel,
        out_type=(
            jax.ShapeDtypeStruct((ep, H), jnp.float32),
            jax.ShapeDtypeStruct((ep, H), jnp.float32),
        ),
        mesh=_mesh,
        scratch_types=[
            pltpu.VMEM((bpw, KB), jnp.int32),
            pltpu.VMEM((bpw, KB), jnp.int32),
            pltpu.VMEM((2, KB, H), jnp.float32),
            pltpu.VMEM((2, KB, H), jnp.float32),
            pltpu.SemaphoreType.DMA,
            pltpu.SemaphoreType.DMA,
            pltpu.SemaphoreType.DMA,
        ],
    )
    def _sc_gather(src_hbm, dst_hbm, h_hbm, hs_hbm, hd_hbm, si_v, di_v,
                   ra_v, rb_v, sem_a, sem_b, sem_w):
        wid = _worker_id()

        pltpu.sync_copy(src_hbm.at[wid], si_v)
        pltpu.sync_copy(dst_hbm.at[wid], di_v)
        pltpu.async_copy(h_hbm.at[si_v.at[0]], ra_v.at[0], sem_a)
        pltpu.async_copy(h_hbm.at[di_v.at[0]], rb_v.at[0], sem_b)

        def batch_body(t, _):
            cur = lax.rem(t, 2)
            nxt = lax.rem(t + 1, 2)
            off = pl.multiple_of((wid * bpw + t) * KB, KB)
            pltpu.make_async_copy(h_hbm.at[si_v.at[0]], ra_v.at[cur],
                                  sem_a).wait()
            pltpu.make_async_copy(h_hbm.at[di_v.at[0]], rb_v.at[cur],
                                  sem_b).wait()

            @pl.when(t + 1 < bpw)
            def _issue():
                @pl.when(t >= 1)
                def _drain():
                    pltpu.make_async_copy(ra_v.at[0],
                                          hs_hbm.at[pl.ds(0, KB)],
                                          sem_w).wait()
                    pltpu.make_async_copy(rb_v.at[0],
                                          hd_hbm.at[pl.ds(0, KB)],
                                          sem_w).wait()

                pltpu.async_copy(h_hbm.at[si_v.at[t + 1]], ra_v.at[nxt],
                                 sem_a)
                pltpu.async_copy(h_hbm.at[di_v.at[t + 1]], rb_v.at[nxt],
                                 sem_b)

            pltpu.async_copy(ra_v.at[cur], hs_hbm.at[pl.ds(off, KB)], sem_w)
            pltpu.async_copy(rb_v.at[cur], hd_hbm.at[pl.ds(off, KB)], sem_w)
            return _

        lax.fori_loop(0, bpw, batch_body, None)
        for _ in range(2):
            pltpu.make_async_copy(ra_v.at[0], hs_hbm.at[pl.ds(0, KB)],
                                  sem_w).wait()
            pltpu.make_async_copy(rb_v.at[0], hd_hbm.at[pl.ds(0, KB)],
                                  sem_w).wait()

    return _sc_gather


_sc_gather = _make_sc_gather(BPW, E)


def _tc_mm_body(degp_ref, x_ref, w_ref, u_ref, dis_ref):
    deg = jnp.sum(degp_ref[...], axis=0)[:N] + 1.0
    dis = (1.0 / jnp.sqrt(deg))[:, None]
    dis_ref[...] = dis
    h = jnp.dot(x_ref[...], w_ref[...], preferred_element_type=jnp.float32)
    u_ref[...] = h * dis


def _tc_mm(degp, x, w):
    return pl.pallas_call(
        _tc_mm_body,
        out_shape=(jax.ShapeDtypeStruct((N, H), jnp.float32),
                   jax.ShapeDtypeStruct((N, 1), jnp.float32)),
    )(degp, x, w)


def _bn_relu(slo_ref, shi_ref, u_ref, dis_ref, g_ref, be_ref):
    Slo = slo_ref[...]
    Shi = shi_ref[...]
    S = jnp.concatenate([Slo[0, :N] + Slo[1, :N], Shi[0, :N] + Shi[1, :N]],
                        axis=-1)
    a = (S + u_ref[...]) * dis_ref[...]
    m = jnp.mean(a, axis=0, keepdims=True)
    v = jnp.mean((a - m) ** 2, axis=0, keepdims=True)
    hn = (a - m) * jax.lax.rsqrt(v + 1e-5) * g_ref[...] + be_ref[...]
    return jnp.maximum(hn, 0.0)


def _tc_bnmm_body(slo_ref, shi_ref, u_ref, dis_ref, g_ref, be_ref, w_ref,
                  o_ref):
    hh = _bn_relu(slo_ref, shi_ref, u_ref, dis_ref, g_ref, be_ref)
    o_ref[...] = jnp.dot(hh, w_ref[...],
                         preferred_element_type=jnp.float32) * dis_ref[...]


def _tc_bnmm(Slo, Shi, u, dis, g, be, w):
    return pl.pallas_call(
        _tc_bnmm_body,
        out_shape=jax.ShapeDtypeStruct((N, H), jnp.float32),
    )(Slo, Shi, u, dis, g, be, w)


def _tc_bn_body(slo_ref, shi_ref, u_ref, dis_ref, g_ref, be_ref, o_ref):
    o_ref[...] = _bn_relu(slo_ref, shi_ref, u_ref, dis_ref, g_ref, be_ref)


def _tc_bn(Slo, Shi, u, dis, g, be):
    return pl.pallas_call(
        _tc_bn_body,
        out_shape=jax.ShapeDtypeStruct((N, H), jnp.float32),
    )(Slo, Shi, u, dis, g, be)


def _bdot(a, b):
    return jnp.dot(a.astype(jnp.bfloat16), b.astype(jnp.bfloat16),
                   preferred_element_type=jnp.float32)


def _tc_mlp_body(hs_ref, hd_ref, ea_ref, wa_ref, wb_ref, we_ref, bc1_ref,
                 w2_ref, bc2_ref, w3_ref, bc3_ref, w4_ref, bc4_ref, o_ref):
    z = (_bdot(hs_ref[...], wa_ref[...])
         + _bdot(hd_ref[...], wb_ref[...])
         + jnp.dot(ea_ref[...], we_ref[...], preferred_element_type=jnp.float32)
         + bc1_ref[...])
    z = jnp.maximum(z, 0.0)
    z = jnp.maximum(_bdot(z, w2_ref[...]) + bc2_ref[...], 0.0)
    z = jnp.maximum(_bdot(z, w3_ref[...]) + bc3_ref[...], 0.0)
    o_ref[...] = jnp.dot(z, w4_ref[...],
                         preferred_element_type=jnp.float32) + bc4_ref[...]


def _tc_mlp(hs, hd, ea, wa, wb, we, bc1, w2, bc2, w3, bc3, w4, bc4, e, eb):
    grid = (e // eb,)
    row_spec = lambda w: pl.BlockSpec((eb, w), lambda i: (i, 0))

    def wspec(a):
        return pl.BlockSpec(a.shape, lambda i: tuple(0 for _ in a.shape))

    return pl.pallas_call(
        _tc_mlp_body,
        grid=grid,
        in_specs=[
            row_spec(H), row_spec(H), row_spec(16),
            wspec(wa), wspec(wb), wspec(we), wspec(bc1),
            wspec(w2), wspec(bc2), wspec(w3), wspec(bc3),
            wspec(w4), wspec(bc4),
        ],
        out_specs=pl.BlockSpec((eb, 2), lambda i: (i, 0)),
        out_shape=jax.ShapeDtypeStruct((e, 2), jnp.float32),
    )(hs, hd, ea, wa, wb, we, bc1, w2, bc2, w3, bc3, w4, bc4)


def kernel(x, edge_index, edge_attr, W1, b1, g1, be1, W2, b2, g2, be2,
           W3, b3, g3, be3, Wc1, bc1, Wc2, bc2, Wc3, bc3, Wc4, bc4):
    src = edge_index[0].reshape(NW, BPW, KB)
    dst = edge_index[1].reshape(NW, BPW, KB)

    degp = _sc_deg(dst)

    u1, dis = _tc_mm(degp, x, W1)
    S1lo, S1hi = _seg_sum(src, dst, u1)
    u2 = _tc_bnmm(S1lo, S1hi, u1, dis, g1, be1, W2)
    S2lo, S2hi = _seg_sum(src, dst, u2)
    u3 = _tc_bnmm(S2lo, S2hi, u2, dis, g2, be2, W3)
    S3lo, S3hi = _seg_sum(src, dst, u3)
    h3 = _tc_bn(S3lo, S3hi, u3, dis, g3, be3)

    Wc1a = Wc1[:H]
    Wc1b = Wc1[H:2 * H]
    Wc1e = Wc1[2 * H:]

    hs, hd = _sc_gather(src, dst, h3)
    return _tc_mlp(hs, hd, edge_attr, Wc1a, Wc1b, Wc1e, bc1,
                   Wc2, bc2, Wc3, bc3, Wc4, bc4, E, 8000)


# speedup vs baseline: 1.2530x; 1.2393x over previous
import functools

import jax
import jax.numpy as jnp
from jax import lax
from jax.experimental import pallas as pl
from jax.experimental.pallas import tpu as pltpu
from jax.experimental.pallas import tpu_sc as plsc

N = 10000
E = 320000
H = 128
NC = 2
NS = 16
NW = NC * NS
KB = 80
BPW = 125
EP = NW * BPW * KB
ROWS_PER_TILE = 632
NP = NS * ROWS_PER_TILE

_mesh = plsc.VectorSubcoreMesh(core_axis_name="c", subcore_axis_name="s")


def _worker_id():
    return lax.axis_index("s") * NC + lax.axis_index("c")


@functools.partial(
    pl.kernel,
    out_type=jax.ShapeDtypeStruct((NW, NP), jnp.float32),
    mesh=_mesh,
    scratch_types=[
        pltpu.VMEM((BPW, KB), jnp.int32),
        pltpu.VMEM((NP,), jnp.float32),
    ],
    compiler_params=pltpu.CompilerParams(needs_layout_passes=False),
)
def _sc_deg(dst_hbm, out_hbm, idx_v, deg_v):
    wid = _worker_id()
    zero16 = jnp.zeros((16,), jnp.float32)
    ones16 = jnp.ones((16,), jnp.float32)

    def zero_body(i, _):
        deg_v[pl.ds(i * 16, 16)] = zero16
        return _

    lax.fori_loop(0, NP // 16, zero_body, None)
    pltpu.sync_copy(dst_hbm.at[wid], idx_v)

    def batch_body(t, _):
        for j in range(KB // 16):
            iv = idx_v[t, pl.ds(j * 16, 16)]
            plsc.addupdate_scatter(deg_v, [iv], ones16)
        return _

    lax.fori_loop(0, BPW, batch_body, None)
    pltpu.sync_copy(deg_v, out_hbm.at[wid])


HW = H // 2


@functools.partial(
    pl.kernel,
    out_type=jax.ShapeDtypeStruct((NC, NP, HW), jnp.float32),
    mesh=_mesh,
    scratch_types=[
        pltpu.VMEM((BPW, KB), jnp.int32),
        pltpu.VMEM((BPW, KB), jnp.int32),
        pltpu.VMEM((3, KB, HW), jnp.float32),
        pltpu.VMEM((ROWS_PER_TILE, HW), jnp.float32),
        pltpu.VMEM_SHARED((NP, HW), jnp.float32),
        pltpu.SemaphoreType.DMA,
        pltpu.SemaphoreType.DMA,
    ],
    compiler_params=pltpu.CompilerParams(use_tc_tiling_on_sc=False),
)
def _sc_seg(src_hbm, dst_hbm, u_hbm, out_hbm, si_v, di_v, rows_v, buf_v,
            acc_sh, sem_g, sem_s):
    c = lax.axis_index("c")
    s = lax.axis_index("s")
    wid = _worker_id()
    zero16 = jnp.zeros((16,), jnp.float32)

    def zero_body(i, _):
        for j in range(HW // 16):
            buf_v[i, pl.ds(j * 16, 16)] = zero16
        return _

    lax.fori_loop(0, ROWS_PER_TILE, zero_body, None)
    pltpu.sync_copy(buf_v, acc_sh.at[pl.ds(s * ROWS_PER_TILE, ROWS_PER_TILE)])
    plsc.subcore_barrier()

    pltpu.sync_copy(src_hbm.at[wid], si_v)
    pltpu.sync_copy(dst_hbm.at[wid], di_v)
    pltpu.async_copy(u_hbm.at[si_v.at[0]], rows_v.at[0], sem_g)
    pltpu.async_copy(u_hbm.at[si_v.at[1]], rows_v.at[1], sem_g)

    def batch_body(t, _):
        cur = lax.rem(t, 3)
        pltpu.make_async_copy(u_hbm.at[si_v.at[0]], rows_v.at[cur],
                              sem_g).wait()

        @pl.when(t + 2 < BPW)
        def _issue():
            @pl.when(t >= 1)
            def _drain():
                pltpu.make_async_copy(rows_v.at[0], acc_sh.at[di_v.at[0]],
                                      sem_s).wait()

            pltpu.async_copy(u_hbm.at[si_v.at[t + 2]],
                             rows_v.at[lax.rem(t + 2, 3)], sem_g)

        pltpu.async_copy(rows_v.at[cur], acc_sh.at[di_v.at[t]], sem_s,
                         add=True)
        return _

    lax.fori_loop(0, BPW, batch_body, None)
    pltpu.make_async_copy(rows_v.at[0], acc_sh.at[di_v.at[0]], sem_s).wait()
    pltpu.make_async_copy(rows_v.at[0], acc_sh.at[di_v.at[0]], sem_s).wait()
    pltpu.make_async_copy(rows_v.at[0], acc_sh.at[di_v.at[0]], sem_s).wait()
    plsc.subcore_barrier()
    pltpu.sync_copy(acc_sh.at[pl.ds(s * ROWS_PER_TILE, ROWS_PER_TILE)], buf_v)
    pltpu.sync_copy(buf_v,
                    out_hbm.at[c, pl.ds(s * ROWS_PER_TILE, ROWS_PER_TILE)])


def _seg_sum(src, dst, u):
    lo = _sc_seg(src, dst, u[:, :HW])
    hi = _sc_seg(src, dst, u[:, HW:])
    return lo, hi


def _make_sc_gather(bpw, ep):
    @functools.partial(
        pl.kernel,
        out_type=(
            jax.ShapeDtypeStruct((ep, H), jnp.float32),
            jax.ShapeDtypeStruct((ep, H), jnp.float32),
        ),
        mesh=_mesh,
        scratch_types=[
            pltpu.VMEM((bpw, KB), jnp.int32),
            pltpu.VMEM((bpw, KB), jnp.int32),
            pltpu.VMEM((2, KB, H), jnp.float32),
            pltpu.VMEM((2, KB, H), jnp.float32),
            pltpu.SemaphoreType.DMA,
            pltpu.SemaphoreType.DMA,
            pltpu.SemaphoreType.DMA,
        ],
    )
    def _sc_gather(src_hbm, dst_hbm, h_hbm, hs_hbm, hd_hbm, si_v, di_v,
                   ra_v, rb_v, sem_a, sem_b, sem_w):
        wid = _worker_id()

        pltpu.sync_copy(src_hbm.at[wid], si_v)
        pltpu.sync_copy(dst_hbm.at[wid], di_v)
        pltpu.async_copy(h_hbm.at[si_v.at[0]], ra_v.at[0], sem_a)
        pltpu.async_copy(h_hbm.at[di_v.at[0]], rb_v.at[0], sem_b)

        def batch_body(t, _):
            cur = lax.rem(t, 2)
            nxt = lax.rem(t + 1, 2)
            off = pl.multiple_of((wid * bpw + t) * KB, KB)
            pltpu.make_async_copy(h_hbm.at[si_v.at[0]], ra_v.at[cur],
                                  sem_a).wait()
            pltpu.make_async_copy(h_hbm.at[di_v.at[0]], rb_v.at[cur],
                                  sem_b).wait()

            @pl.when(t + 1 < bpw)
            def _issue():
                @pl.when(t >= 1)
                def _drain():
                    pltpu.make_async_copy(ra_v.at[0],
                                          hs_hbm.at[pl.ds(0, KB)],
                                          sem_w).wait()
                    pltpu.make_async_copy(rb_v.at[0],
                                          hd_hbm.at[pl.ds(0, KB)],
                                          sem_w).wait()

                pltpu.async_copy(h_hbm.at[si_v.at[t + 1]], ra_v.at[nxt],
                                 sem_a)
                pltpu.async_copy(h_hbm.at[di_v.at[t + 1]], rb_v.at[nxt],
                                 sem_b)

            pltpu.async_copy(ra_v.at[cur], hs_hbm.at[pl.ds(off, KB)], sem_w)
            pltpu.async_copy(rb_v.at[cur], hd_hbm.at[pl.ds(off, KB)], sem_w)
            return _

        lax.fori_loop(0, bpw, batch_body, None)
        for _ in range(2):
            pltpu.make_async_copy(ra_v.at[0], hs_hbm.at[pl.ds(0, KB)],
                                  sem_w).wait()
            pltpu.make_async_copy(rb_v.at[0], hd_hbm.at[pl.ds(0, KB)],
                                  sem_w).wait()

    return _sc_gather


_sc_gather = _make_sc_gather(BPW, E)


def _tc_mm_body(degp_ref, x_ref, w_ref, u_ref, dis_ref):
    deg = jnp.sum(degp_ref[...], axis=0)[:N] + 1.0
    dis = (1.0 / jnp.sqrt(deg))[:, None]
    dis_ref[...] = dis
    h = jnp.dot(x_ref[...], w_ref[...], preferred_element_type=jnp.float32)
    u_ref[...] = h * dis


def _tc_mm(degp, x, w):
    return pl.pallas_call(
        _tc_mm_body,
        out_shape=(jax.ShapeDtypeStruct((N, H), jnp.float32),
                   jax.ShapeDtypeStruct((N, 1), jnp.float32)),
    )(degp, x, w)


def _bn_relu(slo_ref, shi_ref, u_ref, dis_ref, g_ref, be_ref):
    Slo = slo_ref[...]
    Shi = shi_ref[...]
    S = jnp.concatenate([Slo[0, :N] + Slo[1, :N], Shi[0, :N] + Shi[1, :N]],
                        axis=-1)
    a = (S + u_ref[...]) * dis_ref[...]
    m = jnp.mean(a, axis=0, keepdims=True)
    v = jnp.mean((a - m) ** 2, axis=0, keepdims=True)
    hn = (a - m) * jax.lax.rsqrt(v + 1e-5) * g_ref[...] + be_ref[...]
    return jnp.maximum(hn, 0.0)


def _tc_bnmm_body(slo_ref, shi_ref, u_ref, dis_ref, g_ref, be_ref, w_ref,
                  o_ref):
    hh = _bn_relu(slo_ref, shi_ref, u_ref, dis_ref, g_ref, be_ref)
    o_ref[...] = jnp.dot(hh, w_ref[...],
                         preferred_element_type=jnp.float32) * dis_ref[...]


def _tc_bnmm(Slo, Shi, u, dis, g, be, w):
    return pl.pallas_call(
        _tc_bnmm_body,
        out_shape=jax.ShapeDtypeStruct((N, H), jnp.float32),
    )(Slo, Shi, u, dis, g, be, w)


def _tc_bn_body(slo_ref, shi_ref, u_ref, dis_ref, g_ref, be_ref, o_ref):
    o_ref[...] = _bn_relu(slo_ref, shi_ref, u_ref, dis_ref, g_ref, be_ref)


def _tc_bn(Slo, Shi, u, dis, g, be):
    return pl.pallas_call(
        _tc_bn_body,
        out_shape=jax.ShapeDtypeStruct((N, H), jnp.float32),
    )(Slo, Shi, u, dis, g, be)


def _bdot(a, b):
    return jnp.dot(a.astype(jnp.bfloat16), b.astype(jnp.bfloat16),
                   preferred_element_type=jnp.float32)


def _tc_mlp_body(hs_ref, hd_ref, ea_ref, wa_ref, wb_ref, we_ref, bc1_ref,
                 w2_ref, bc2_ref, w3_ref, bc3_ref, w4_ref, bc4_ref, o_ref):
    z = (_bdot(hs_ref[...], wa_ref[...])
         + _bdot(hd_ref[...], wb_ref[...])
         + jnp.dot(ea_ref[...], we_ref[...], preferred_element_type=jnp.float32)
         + bc1_ref[...])
    z = jnp.maximum(z, 0.0)
    z = jnp.maximum(_bdot(z, w2_ref[...]) + bc2_ref[...], 0.0)
    z = jnp.maximum(_bdot(z, w3_ref[...]) + bc3_ref[...], 0.0)
    o_ref[...] = jnp.dot(z, w4_ref[...],
                         preferred_element_type=jnp.float32) + bc4_ref[...]


def _tc_mlp(hs, hd, ea, wa, wb, we, bc1, w2, bc2, w3, bc3, w4, bc4, e, eb):
    grid = (e // eb,)
    row_spec = lambda w: pl.BlockSpec((eb, w), lambda i: (i, 0))

    def wspec(a):
        return pl.BlockSpec(a.shape, lambda i: tuple(0 for _ in a.shape))

    return pl.pallas_call(
        _tc_mlp_body,
        grid=grid,
        in_specs=[
            row_spec(H), row_spec(H), row_spec(16),
            wspec(wa), wspec(wb), wspec(we), wspec(bc1),
            wspec(w2), wspec(bc2), wspec(w3), wspec(bc3),
            wspec(w4), wspec(bc4),
        ],
        out_specs=pl.BlockSpec((eb, 2), lambda i: (i, 0)),
        out_shape=jax.ShapeDtypeStruct((e, 2), jnp.float32),
    )(hs, hd, ea, wa, wb, we, bc1, w2, bc2, w3, bc3, w4, bc4)


def kernel(x, edge_index, edge_attr, W1, b1, g1, be1, W2, b2, g2, be2,
           W3, b3, g3, be3, Wc1, bc1, Wc2, bc2, Wc3, bc3, Wc4, bc4):
    src = edge_index[0].reshape(NW, BPW, KB)
    dst = edge_index[1].reshape(NW, BPW, KB)

    degp = _sc_deg(dst)

    u1, dis = _tc_mm(degp, x, W1)
    S1lo, S1hi = _seg_sum(src, dst, u1)
    u2 = _tc_bnmm(S1lo, S1hi, u1, dis, g1, be1, W2)
    S2lo, S2hi = _seg_sum(src, dst, u2)
    u3 = _tc_bnmm(S2lo, S2hi, u2, dis, g2, be2, W3)
    S3lo, S3hi = _seg_sum(src, dst, u3)
    h3 = _tc_bn(S3lo, S3hi, u3, dis, g3, be3)

    Wc1a = Wc1[:H]
    Wc1b = Wc1[H:2 * H]
    Wc1e = Wc1[2 * H:]

    hs, hd = _sc_gather(src, dst, h3)
    return _tc_mlp(hs, hd, edge_attr, Wc1a, Wc1b, Wc1e, bc1,
                   Wc2, bc2, Wc3, bc3, Wc4, bc4, E, 8000)


# speedup vs baseline: 1.2618x; 1.0070x over previous
import functools

import jax
import jax.numpy as jnp
from jax import lax
from jax.experimental import pallas as pl
from jax.experimental.pallas import tpu as pltpu
from jax.experimental.pallas import tpu_sc as plsc

N = 10000
E = 320000
H = 128
NC = 2
NS = 16
NW = NC * NS
KB = 80
BPW = 125
EP = NW * BPW * KB
ROWS_PER_TILE = 632
NP = NS * ROWS_PER_TILE

_mesh = plsc.VectorSubcoreMesh(core_axis_name="c", subcore_axis_name="s")


def _worker_id():
    return lax.axis_index("s") * NC + lax.axis_index("c")


@functools.partial(
    pl.kernel,
    out_type=jax.ShapeDtypeStruct((NW, NP), jnp.float32),
    mesh=_mesh,
    scratch_types=[
        pltpu.VMEM((BPW, KB), jnp.int32),
        pltpu.VMEM((NP,), jnp.float32),
    ],
    compiler_params=pltpu.CompilerParams(needs_layout_passes=False),
)
def _sc_deg(dst_hbm, out_hbm, idx_v, deg_v):
    wid = _worker_id()
    zero16 = jnp.zeros((16,), jnp.float32)
    ones16 = jnp.ones((16,), jnp.float32)

    def zero_body(i, _):
        deg_v[pl.ds(i * 16, 16)] = zero16
        return _

    lax.fori_loop(0, NP // 16, zero_body, None)
    pltpu.sync_copy(dst_hbm.at[wid], idx_v)

    def batch_body(t, _):
        for j in range(KB // 16):
            iv = idx_v[t, pl.ds(j * 16, 16)]
            plsc.addupdate_scatter(deg_v, [iv], ones16)
        return _

    lax.fori_loop(0, BPW, batch_body, None)
    pltpu.sync_copy(deg_v, out_hbm.at[wid])


HW = H // 2


@functools.partial(
    pl.kernel,
    out_type=jax.ShapeDtypeStruct((NC, NP, HW), jnp.float32),
    mesh=_mesh,
    scratch_types=[
        pltpu.VMEM((BPW, KB), jnp.int32),
        pltpu.VMEM((BPW, KB), jnp.int32),
        pltpu.VMEM((3, KB, HW), jnp.float32),
        pltpu.VMEM((ROWS_PER_TILE, HW), jnp.float32),
        pltpu.VMEM_SHARED((NP, HW), jnp.float32),
        pltpu.SemaphoreType.DMA,
        pltpu.SemaphoreType.DMA,
    ],
    compiler_params=pltpu.CompilerParams(use_tc_tiling_on_sc=False),
)
def _sc_seg(src_hbm, dst_hbm, u_hbm, out_hbm, si_v, di_v, rows_v, buf_v,
            acc_sh, sem_g, sem_s):
    c = lax.axis_index("c")
    s = lax.axis_index("s")
    wid = _worker_id()
    zero16 = jnp.zeros((16,), jnp.float32)

    def zero_body(i, _):
        for j in range(HW // 16):
            buf_v[i, pl.ds(j * 16, 16)] = zero16
        return _

    lax.fori_loop(0, ROWS_PER_TILE, zero_body, None)
    pltpu.sync_copy(buf_v, acc_sh.at[pl.ds(s * ROWS_PER_TILE, ROWS_PER_TILE)])
    plsc.subcore_barrier()

    pltpu.sync_copy(src_hbm.at[wid], si_v)
    pltpu.sync_copy(dst_hbm.at[wid], di_v)
    pltpu.async_copy(u_hbm.at[si_v.at[0]], rows_v.at[0], sem_g)
    pltpu.async_copy(u_hbm.at[si_v.at[1]], rows_v.at[1], sem_g)

    def batch_body(t, _):
        cur = lax.rem(t, 3)
        pltpu.make_async_copy(u_hbm.at[si_v.at[0]], rows_v.at[cur],
                              sem_g).wait()

        @pl.when(t + 2 < BPW)
        def _issue():
            @pl.when(t >= 1)
            def _drain():
                pltpu.make_async_copy(rows_v.at[0], acc_sh.at[di_v.at[0]],
                                      sem_s).wait()

            pltpu.async_copy(u_hbm.at[si_v.at[t + 2]],
                             rows_v.at[lax.rem(t + 2, 3)], sem_g)

        pltpu.async_copy(rows_v.at[cur], acc_sh.at[di_v.at[t]], sem_s,
                         add=True)
        return _

    lax.fori_loop(0, BPW, batch_body, None)
    pltpu.make_async_copy(rows_v.at[0], acc_sh.at[di_v.at[0]], sem_s).wait()
    pltpu.make_async_copy(rows_v.at[0], acc_sh.at[di_v.at[0]], sem_s).wait()
    pltpu.make_async_copy(rows_v.at[0], acc_sh.at[di_v.at[0]], sem_s).wait()
    plsc.subcore_barrier()
    pltpu.sync_copy(acc_sh.at[pl.ds(s * ROWS_PER_TILE, ROWS_PER_TILE)], buf_v)
    pltpu.sync_copy(buf_v,
                    out_hbm.at[c, pl.ds(s * ROWS_PER_TILE, ROWS_PER_TILE)])


def _seg_sum(src, dst, u):
    lo = _sc_seg(src, dst, u[:, :HW])
    hi = _sc_seg(src, dst, u[:, HW:])
    return lo, hi


def _make_sc_gather(bpw, ep):
    @functools.partial(
        pl.kernel,
        out_type=(
            jax.ShapeDtypeStruct((ep, H), jnp.float32),
            jax.ShapeDtypeStruct((ep, H), jnp.float32),
        ),
        mesh=_mesh,
        scratch_types=[
            pltpu.VMEM((bpw, KB), jnp.int32),
            pltpu.VMEM((bpw, KB), jnp.int32),
            pltpu.VMEM((3, KB, H), jnp.float32),
            pltpu.VMEM((3, KB, H), jnp.float32),
            pltpu.SemaphoreType.DMA,
            pltpu.SemaphoreType.DMA,
            pltpu.SemaphoreType.DMA,
        ],
    )
    def _sc_gather(src_hbm, dst_hbm, h_hbm, hs_hbm, hd_hbm, si_v, di_v,
                   ra_v, rb_v, sem_a, sem_b, sem_w):
        wid = _worker_id()

        pltpu.sync_copy(src_hbm.at[wid], si_v)
        pltpu.sync_copy(dst_hbm.at[wid], di_v)
        for p in range(2):
            pltpu.async_copy(h_hbm.at[si_v.at[p]], ra_v.at[p], sem_a)
            pltpu.async_copy(h_hbm.at[di_v.at[p]], rb_v.at[p], sem_b)

        def batch_body(t, _):
            cur = lax.rem(t, 3)
            off = pl.multiple_of((wid * bpw + t) * KB, KB)
            pltpu.make_async_copy(h_hbm.at[si_v.at[0]], ra_v.at[cur],
                                  sem_a).wait()
            pltpu.make_async_copy(h_hbm.at[di_v.at[0]], rb_v.at[cur],
                                  sem_b).wait()

            @pl.when(t + 2 < bpw)
            def _issue():
                @pl.when(t >= 1)
                def _drain():
                    pltpu.make_async_copy(ra_v.at[0],
                                          hs_hbm.at[pl.ds(0, KB)],
                                          sem_w).wait()
                    pltpu.make_async_copy(rb_v.at[0],
                                          hd_hbm.at[pl.ds(0, KB)],
                                          sem_w).wait()

                nxt = lax.rem(t + 2, 3)
                pltpu.async_copy(h_hbm.at[si_v.at[t + 2]], ra_v.at[nxt],
                                 sem_a)
                pltpu.async_copy(h_hbm.at[di_v.at[t + 2]], rb_v.at[nxt],
                                 sem_b)

            pltpu.async_copy(ra_v.at[cur], hs_hbm.at[pl.ds(off, KB)], sem_w)
            pltpu.async_copy(rb_v.at[cur], hd_hbm.at[pl.ds(off, KB)], sem_w)
            return _

        lax.fori_loop(0, bpw, batch_body, None)
        for _ in range(3):
            pltpu.make_async_copy(ra_v.at[0], hs_hbm.at[pl.ds(0, KB)],
                                  sem_w).wait()
            pltpu.make_async_copy(rb_v.at[0], hd_hbm.at[pl.ds(0, KB)],
                                  sem_w).wait()

    return _sc_gather


_sc_gather = _make_sc_gather(BPW, E)


def _tc_mm_body(degp_ref, x_ref, w_ref, u_ref, dis_ref):
    deg = jnp.sum(degp_ref[...], axis=0)[:N] + 1.0
    dis = (1.0 / jnp.sqrt(deg))[:, None]
    dis_ref[...] = dis
    h = jnp.dot(x_ref[...], w_ref[...], preferred_element_type=jnp.float32)
    u_ref[...] = h * dis


def _tc_mm(degp, x, w):
    return pl.pallas_call(
        _tc_mm_body,
        out_shape=(jax.ShapeDtypeStruct((N, H), jnp.float32),
                   jax.ShapeDtypeStruct((N, 1), jnp.float32)),
    )(degp, x, w)


def _bn_relu(slo_ref, shi_ref, u_ref, dis_ref, g_ref, be_ref):
    Slo = slo_ref[...]
    Shi = shi_ref[...]
    S = jnp.concatenate([Slo[0, :N] + Slo[1, :N], Shi[0, :N] + Shi[1, :N]],
                        axis=-1)
    a = (S + u_ref[...]) * dis_ref[...]
    m = jnp.mean(a, axis=0, keepdims=True)
    v = jnp.mean((a - m) ** 2, axis=0, keepdims=True)
    hn = (a - m) * jax.lax.rsqrt(v + 1e-5) * g_ref[...] + be_ref[...]
    return jnp.maximum(hn, 0.0)


def _tc_bnmm_body(slo_ref, shi_ref, u_ref, dis_ref, g_ref, be_ref, w_ref,
                  o_ref):
    hh = _bn_relu(slo_ref, shi_ref, u_ref, dis_ref, g_ref, be_ref)
    o_ref[...] = jnp.dot(hh, w_ref[...],
                         preferred_element_type=jnp.float32) * dis_ref[...]


def _tc_bnmm(Slo, Shi, u, dis, g, be, w):
    return pl.pallas_call(
        _tc_bnmm_body,
        out_shape=jax.ShapeDtypeStruct((N, H), jnp.float32),
    )(Slo, Shi, u, dis, g, be, w)


def _tc_bn_body(slo_ref, shi_ref, u_ref, dis_ref, g_ref, be_ref, o_ref):
    o_ref[...] = _bn_relu(slo_ref, shi_ref, u_ref, dis_ref, g_ref, be_ref)


def _tc_bn(Slo, Shi, u, dis, g, be):
    return pl.pallas_call(
        _tc_bn_body,
        out_shape=jax.ShapeDtypeStruct((N, H), jnp.float32),
    )(Slo, Shi, u, dis, g, be)


def _bdot(a, b):
    return jnp.dot(a.astype(jnp.bfloat16), b.astype(jnp.bfloat16),
                   preferred_element_type=jnp.float32)


def _tc_mlp_body(hs_ref, hd_ref, ea_ref, wa_ref, wb_ref, we_ref, bc1_ref,
                 w2_ref, bc2_ref, w3_ref, bc3_ref, w4_ref, bc4_ref, o_ref):
    z = (_bdot(hs_ref[...], wa_ref[...])
         + _bdot(hd_ref[...], wb_ref[...])
         + jnp.dot(ea_ref[...], we_ref[...], preferred_element_type=jnp.float32)
         + bc1_ref[...])
    z = jnp.maximum(z, 0.0)
    z = jnp.maximum(_bdot(z, w2_ref[...]) + bc2_ref[...], 0.0)
    z = jnp.maximum(_bdot(z, w3_ref[...]) + bc3_ref[...], 0.0)
    o_ref[...] = jnp.dot(z, w4_ref[...],
                         preferred_element_type=jnp.float32) + bc4_ref[...]


def _tc_mlp(hs, hd, ea, wa, wb, we, bc1, w2, bc2, w3, bc3, w4, bc4, e, eb):
    grid = (e // eb,)
    row_spec = lambda w: pl.BlockSpec((eb, w), lambda i: (i, 0))

    def wspec(a):
        return pl.BlockSpec(a.shape, lambda i: tuple(0 for _ in a.shape))

    return pl.pallas_call(
        _tc_mlp_body,
        grid=grid,
        in_specs=[
            row_spec(H), row_spec(H), row_spec(16),
            wspec(wa), wspec(wb), wspec(we), wspec(bc1),
            wspec(w2), wspec(bc2), wspec(w3), wspec(bc3),
            wspec(w4), wspec(bc4),
        ],
        out_specs=pl.BlockSpec((eb, 2), lambda i: (i, 0)),
        out_shape=jax.ShapeDtypeStruct((e, 2), jnp.float32),
    )(hs, hd, ea, wa, wb, we, bc1, w2, bc2, w3, bc3, w4, bc4)


def kernel(x, edge_index, edge_attr, W1, b1, g1, be1, W2, b2, g2, be2,
           W3, b3, g3, be3, Wc1, bc1, Wc2, bc2, Wc3, bc3, Wc4, bc4):
    src = edge_index[0].reshape(NW, BPW, KB)
    dst = edge_index[1].reshape(NW, BPW, KB)

    degp = _sc_deg(dst)

    u1, dis = _tc_mm(degp, x, W1)
    S1lo, S1hi = _seg_sum(src, dst, u1)
    u2 = _tc_bnmm(S1lo, S1hi, u1, dis, g1, be1, W2)
    S2lo, S2hi = _seg_sum(src, dst, u2)
    u3 = _tc_bnmm(S2lo, S2hi, u2, dis, g2, be2, W3)
    S3lo, S3hi = _seg_sum(src, dst, u3)
    h3 = _tc_bn(S3lo, S3hi, u3, dis, g3, be3)

    Wc1a = Wc1[:H]
    Wc1b = Wc1[H:2 * H]
    Wc1e = Wc1[2 * H:]

    hs, hd = _sc_gather(src, dst, h3)
    return _tc_mlp(hs, hd, edge_attr, Wc1a, Wc1b, Wc1e, bc1,
                   Wc2, bc2, Wc3, bc3, Wc4, bc4, E, 8000)


# speedup vs baseline: 1.3536x; 1.0727x over previous
import functools

import jax
import jax.numpy as jnp
from jax import lax
from jax.experimental import pallas as pl
from jax.experimental.pallas import tpu as pltpu
from jax.experimental.pallas import tpu_sc as plsc

N = 10000
E = 320000
H = 128
NC = 2
NS = 16
NW = NC * NS
KB = 80
BPW = 125
EP = NW * BPW * KB
ROWS_PER_TILE = 632
NP = NS * ROWS_PER_TILE

_mesh = plsc.VectorSubcoreMesh(core_axis_name="c", subcore_axis_name="s")


def _worker_id():
    return lax.axis_index("s") * NC + lax.axis_index("c")


@functools.partial(
    pl.kernel,
    out_type=jax.ShapeDtypeStruct((NW, NP), jnp.float32),
    mesh=_mesh,
    scratch_types=[
        pltpu.VMEM((BPW, KB), jnp.int32),
        pltpu.VMEM((NP,), jnp.float32),
    ],
    compiler_params=pltpu.CompilerParams(needs_layout_passes=False),
)
def _sc_deg(dst_hbm, out_hbm, idx_v, deg_v):
    wid = _worker_id()
    zero16 = jnp.zeros((16,), jnp.float32)
    ones16 = jnp.ones((16,), jnp.float32)

    def zero_body(i, _):
        deg_v[pl.ds(i * 16, 16)] = zero16
        return _

    lax.fori_loop(0, NP // 16, zero_body, None)
    pltpu.sync_copy(dst_hbm.at[wid], idx_v)

    def batch_body(t, _):
        for j in range(KB // 16):
            iv = idx_v[t, pl.ds(j * 16, 16)]
            plsc.addupdate_scatter(deg_v, [iv], ones16)
        return _

    lax.fori_loop(0, BPW, batch_body, None)
    pltpu.sync_copy(deg_v, out_hbm.at[wid])


HW = H // 2


@functools.partial(
    pl.kernel,
    out_type=jax.ShapeDtypeStruct((NC, NP, HW), jnp.float32),
    mesh=_mesh,
    scratch_types=[
        pltpu.VMEM((BPW, KB), jnp.int32),
        pltpu.VMEM((BPW, KB), jnp.int32),
        pltpu.VMEM((4, KB, HW), jnp.float32),
        pltpu.VMEM((ROWS_PER_TILE, HW), jnp.float32),
        pltpu.VMEM_SHARED((NP, HW), jnp.float32),
        pltpu.SemaphoreType.DMA,
        pltpu.SemaphoreType.DMA,
    ],
    compiler_params=pltpu.CompilerParams(use_tc_tiling_on_sc=False),
)
def _sc_seg(src_hbm, dst_hbm, u_hbm, out_hbm, si_v, di_v, rows_v, buf_v,
            acc_sh, sem_g, sem_s):
    c = lax.axis_index("c")
    s = lax.axis_index("s")
    wid = _worker_id()
    zero16 = jnp.zeros((16,), jnp.float32)

    def zero_body(i, _):
        for j in range(HW // 16):
            buf_v[i, pl.ds(j * 16, 16)] = zero16
        return _

    lax.fori_loop(0, ROWS_PER_TILE, zero_body, None)
    pltpu.sync_copy(buf_v, acc_sh.at[pl.ds(s * ROWS_PER_TILE, ROWS_PER_TILE)])
    plsc.subcore_barrier()

    pltpu.sync_copy(src_hbm.at[wid], si_v)
    pltpu.sync_copy(dst_hbm.at[wid], di_v)
    for p in range(3):
        pltpu.async_copy(u_hbm.at[si_v.at[p]], rows_v.at[p], sem_g)

    def batch_body(t, _):
        cur = lax.rem(t, 4)
        pltpu.make_async_copy(u_hbm.at[si_v.at[0]], rows_v.at[cur],
                              sem_g).wait()

        @pl.when(t + 3 < BPW)
        def _issue():
            @pl.when(t >= 1)
            def _drain():
                pltpu.make_async_copy(rows_v.at[0], acc_sh.at[di_v.at[0]],
                                      sem_s).wait()

            pltpu.async_copy(u_hbm.at[si_v.at[t + 3]],
                             rows_v.at[lax.rem(t + 3, 4)], sem_g)

        pltpu.async_copy(rows_v.at[cur], acc_sh.at[di_v.at[t]], sem_s,
                         add=True)
        return _

    lax.fori_loop(0, BPW, batch_body, None)
    for _ in range(4):
        pltpu.make_async_copy(rows_v.at[0], acc_sh.at[di_v.at[0]],
                              sem_s).wait()
    plsc.subcore_barrier()
    pltpu.sync_copy(acc_sh.at[pl.ds(s * ROWS_PER_TILE, ROWS_PER_TILE)], buf_v)
    pltpu.sync_copy(buf_v,
                    out_hbm.at[c, pl.ds(s * ROWS_PER_TILE, ROWS_PER_TILE)])


def _seg_sum(src, dst, u):
    lo = _sc_seg(src, dst, u[:, :HW])
    hi = _sc_seg(src, dst, u[:, HW:])
    return lo, hi


def _make_sc_gather(bpw, ep):
    @functools.partial(
        pl.kernel,
        out_type=(
            jax.ShapeDtypeStruct((ep, H), jnp.float32),
            jax.ShapeDtypeStruct((ep, H), jnp.float32),
        ),
        mesh=_mesh,
        scratch_types=[
            pltpu.VMEM((bpw, KB), jnp.int32),
            pltpu.VMEM((bpw, KB), jnp.int32),
            pltpu.VMEM((3, KB, H), jnp.float32),
            pltpu.VMEM((3, KB, H), jnp.float32),
            pltpu.SemaphoreType.DMA,
            pltpu.SemaphoreType.DMA,
            pltpu.SemaphoreType.DMA,
        ],
    )
    def _sc_gather(src_hbm, dst_hbm, h_hbm, hs_hbm, hd_hbm, si_v, di_v,
                   ra_v, rb_v, sem_a, sem_b, sem_w):
        wid = _worker_id()

        pltpu.sync_copy(src_hbm.at[wid], si_v)
        pltpu.sync_copy(dst_hbm.at[wid], di_v)
        for p in range(2):
            pltpu.async_copy(h_hbm.at[si_v.at[p]], ra_v.at[p], sem_a)
            pltpu.async_copy(h_hbm.at[di_v.at[p]], rb_v.at[p], sem_b)

        def batch_body(t, _):
            cur = lax.rem(t, 3)
            off = pl.multiple_of((wid * bpw + t) * KB, KB)
            pltpu.make_async_copy(h_hbm.at[si_v.at[0]], ra_v.at[cur],
                                  sem_a).wait()
            pltpu.make_async_copy(h_hbm.at[di_v.at[0]], rb_v.at[cur],
                                  sem_b).wait()

            @pl.when(t + 2 < bpw)
            def _issue():
                @pl.when(t >= 1)
                def _drain():
                    pltpu.make_async_copy(ra_v.at[0],
                                          hs_hbm.at[pl.ds(0, KB)],
                                          sem_w).wait()
                    pltpu.make_async_copy(rb_v.at[0],
                                          hd_hbm.at[pl.ds(0, KB)],
                                          sem_w).wait()

                nxt = lax.rem(t + 2, 3)
                pltpu.async_copy(h_hbm.at[si_v.at[t + 2]], ra_v.at[nxt],
                                 sem_a)
                pltpu.async_copy(h_hbm.at[di_v.at[t + 2]], rb_v.at[nxt],
                                 sem_b)

            pltpu.async_copy(ra_v.at[cur], hs_hbm.at[pl.ds(off, KB)], sem_w)
            pltpu.async_copy(rb_v.at[cur], hd_hbm.at[pl.ds(off, KB)], sem_w)
            return _

        lax.fori_loop(0, bpw, batch_body, None)
        for _ in range(3):
            pltpu.make_async_copy(ra_v.at[0], hs_hbm.at[pl.ds(0, KB)],
                                  sem_w).wait()
            pltpu.make_async_copy(rb_v.at[0], hd_hbm.at[pl.ds(0, KB)],
                                  sem_w).wait()

    return _sc_gather


_sc_gather = _make_sc_gather(BPW, E)


def _tc_mm_body(degp_ref, x_ref, w_ref, u_ref, dis_ref):
    deg = jnp.sum(degp_ref[...], axis=0)[:N] + 1.0
    dis = (1.0 / jnp.sqrt(deg))[:, None]
    dis_ref[...] = dis
    h = jnp.dot(x_ref[...], w_ref[...], preferred_element_type=jnp.float32)
    u_ref[...] = h * dis


def _tc_mm(degp, x, w):
    return pl.pallas_call(
        _tc_mm_body,
        out_shape=(jax.ShapeDtypeStruct((N, H), jnp.float32),
                   jax.ShapeDtypeStruct((N, 1), jnp.float32)),
    )(degp, x, w)


def _bn_relu(slo_ref, shi_ref, u_ref, dis_ref, g_ref, be_ref):
    Slo = slo_ref[...]
    Shi = shi_ref[...]
    S = jnp.concatenate([Slo[0, :N] + Slo[1, :N], Shi[0, :N] + Shi[1, :N]],
                        axis=-1)
    a = (S + u_ref[...]) * dis_ref[...]
    m = jnp.mean(a, axis=0, keepdims=True)
    v = jnp.mean((a - m) ** 2, axis=0, keepdims=True)
    hn = (a - m) * jax.lax.rsqrt(v + 1e-5) * g_ref[...] + be_ref[...]
    return jnp.maximum(hn, 0.0)


def _tc_bnmm_body(slo_ref, shi_ref, u_ref, dis_ref, g_ref, be_ref, w_ref,
                  o_ref):
    hh = _bn_relu(slo_ref, shi_ref, u_ref, dis_ref, g_ref, be_ref)
    o_ref[...] = jnp.dot(hh, w_ref[...],
                         preferred_element_type=jnp.float32) * dis_ref[...]


def _tc_bnmm(Slo, Shi, u, dis, g, be, w):
    return pl.pallas_call(
        _tc_bnmm_body,
        out_shape=jax.ShapeDtypeStruct((N, H), jnp.float32),
    )(Slo, Shi, u, dis, g, be, w)


def _tc_bn_body(slo_ref, shi_ref, u_ref, dis_ref, g_ref, be_ref, o_ref):
    o_ref[...] = _bn_relu(slo_ref, shi_ref, u_ref, dis_ref, g_ref, be_ref)


def _tc_bn(Slo, Shi, u, dis, g, be):
    return pl.pallas_call(
        _tc_bn_body,
        out_shape=jax.ShapeDtypeStruct((N, H), jnp.float32),
    )(Slo, Shi, u, dis, g, be)


def _bdot(a, b):
    return jnp.dot(a.astype(jnp.bfloat16), b.astype(jnp.bfloat16),
                   preferred_element_type=jnp.float32)


def _tc_mlp_body(hs_ref, hd_ref, ea_ref, wa_ref, wb_ref, we_ref, bc1_ref,
                 w2_ref, bc2_ref, w3_ref, bc3_ref, w4_ref, bc4_ref, o_ref):
    z = (_bdot(hs_ref[...], wa_ref[...])
         + _bdot(hd_ref[...], wb_ref[...])
         + jnp.dot(ea_ref[...], we_ref[...], preferred_element_type=jnp.float32)
         + bc1_ref[...])
    z = jnp.maximum(z, 0.0)
    z = jnp.maximum(_bdot(z, w2_ref[...]) + bc2_ref[...], 0.0)
    z = jnp.maximum(_bdot(z, w3_ref[...]) + bc3_ref[...], 0.0)
    o_ref[...] = jnp.dot(z, w4_ref[...],
                         preferred_element_type=jnp.float32) + bc4_ref[...]


def _tc_mlp(hs, hd, ea, wa, wb, we, bc1, w2, bc2, w3, bc3, w4, bc4, e, eb):
    grid = (e // eb,)
    row_spec = lambda w: pl.BlockSpec((eb, w), lambda i: (i, 0))

    def wspec(a):
        return pl.BlockSpec(a.shape, lambda i: tuple(0 for _ in a.shape))

    return pl.pallas_call(
        _tc_mlp_body,
        grid=grid,
        in_specs=[
            row_spec(H), row_spec(H), row_spec(16),
            wspec(wa), wspec(wb), wspec(we), wspec(bc1),
            wspec(w2), wspec(bc2), wspec(w3), wspec(bc3),
            wspec(w4), wspec(bc4),
        ],
        out_specs=pl.BlockSpec((eb, 2), lambda i: (i, 0)),
        out_shape=jax.ShapeDtypeStruct((e, 2), jnp.float32),
    )(hs, hd, ea, wa, wb, we, bc1, w2, bc2, w3, bc3, w4, bc4)


def kernel(x, edge_index, edge_attr, W1, b1, g1, be1, W2, b2, g2, be2,
           W3, b3, g3, be3, Wc1, bc1, Wc2, bc2, Wc3, bc3, Wc4, bc4):
    src = edge_index[0].reshape(NW, BPW, KB)
    dst = edge_index[1].reshape(NW, BPW, KB)

    degp = _sc_deg(dst)

    u1, dis = _tc_mm(degp, x, W1)
    S1lo, S1hi = _seg_sum(src, dst, u1)
    u2 = _tc_bnmm(S1lo, S1hi, u1, dis, g1, be1, W2)
    S2lo, S2hi = _seg_sum(src, dst, u2)
    u3 = _tc_bnmm(S2lo, S2hi, u2, dis, g2, be2, W3)
    S3lo, S3hi = _seg_sum(src, dst, u3)
    h3 = _tc_bn(S3lo, S3hi, u3, dis, g3, be3)

    Wc1a = Wc1[:H]
    Wc1b = Wc1[H:2 * H]
    Wc1e = Wc1[2 * H:]

    hs, hd = _sc_gather(src, dst, h3)
    return _tc_mlp(hs, hd, edge_attr, Wc1a, Wc1b, Wc1e, bc1,
                   Wc2, bc2, Wc3, bc3, Wc4, bc4, E, 8000)
